# baseline (speedup 1.0000x reference)
import functools

import jax
import jax.numpy as jnp
from jax import lax
from jax.experimental import pallas as pl
from jax.experimental.pallas import tpu as pltpu
from jax.experimental.pallas import tpu_sc as plsc

_R_MAX = 6.0
_P = 3.0

_NC = 2
_NS = 16
_NW = _NC * _NS


def _message_body(w_ref, pair_ref, upd_ref, dist_ref, out_ref):
    bn = dist_ref.shape[0]
    k = pair_ref.shape[0] // bn
    z = jnp.dot(pair_ref[...], w_ref[...], preferred_element_type=jnp.float32)
    gate = 1.0 / (1.0 + jnp.exp(-z))
    d = dist_ref[...]
    t = d * (1.0 / _R_MAX)
    t_p = t * t * t
    env = (
        1.0
        - (_P + 1.0) * (_P + 2.0) / 2.0 * t_p
        + _P * (_P + 2.0) * t_p * t
        - _P * (_P + 1.0) / 2.0 * t_p * t * t
    )
    env = jnp.where(d < _R_MAX, env, 0.0)
    env_rows = lax.broadcast_in_dim(env, (bn, k, 1), (0, 2)).reshape(bn * k, 1)
    out_ref[...] = gate * upd_ref[...] * env_rows


def _compute_message(pairf, updf, dist2, W, K, BM, m_out, off_blocks):
    D = pairf.shape[1]
    S = W.shape[1]
    BN = BM // K
    grid = (m_out // BM,)
    return pl.pallas_call(
        _message_body,
        grid=grid,
        in_specs=[
            pl.BlockSpec((D, S), lambda i: (0, 0)),
            pl.BlockSpec((BM, D), lambda i: (i + off_blocks, 0)),
            pl.BlockSpec((BM, S), lambda i: (i + off_blocks, 0)),
            pl.BlockSpec((BN, 1), lambda i: (i + off_blocks, 0)),
        ],
        out_specs=pl.BlockSpec((BM, S), lambda i: (i, 0)),
        out_shape=jax.ShapeDtypeStruct((m_out, S), jnp.float32),
        compiler_params=pltpu.CompilerParams(
            dimension_semantics=("arbitrary",),
        ),
    )(W, pairf, updf, dist2)


def _scatter_pass(msg, neigh3, aux, n_nodes, chunk, max_chunks, first, last):
    M, S = msg.shape
    n_acc = n_nodes + 8
    e_w = M // _NW
    n_chunks = e_w // chunk
    rows_half = n_nodes // 2

    mesh = plsc.VectorSubcoreMesh(
        core_axis_name="c", subcore_axis_name="s",
        num_cores=_NC, num_subcores=_NS,
    )

    if last:
        out_type = jax.ShapeDtypeStruct((_NC * n_nodes, S), jnp.float32)
    else:
        out_type = jax.ShapeDtypeStruct((8, S), jnp.float32)

    @functools.partial(
        pl.kernel,
        out_type=out_type,
        mesh=mesh,
        scratch_types=[
            pltpu.VMEM_SHARED((n_acc, S), jnp.float32),
            pltpu.VMEM((max_chunks, chunk), jnp.int32),
            pltpu.VMEM((chunk, S), jnp.float32),
            pltpu.VMEM((chunk, S), jnp.float32),
            pltpu.SemaphoreType.DMA,
            pltpu.SemaphoreType.DMA,
        ],
    )
    def k(msg_hbm, idx_hbm, aux_hbm, out_hbm, acc_sh, idx_v, buf0, buf1,
          sem0, sem1):
        cid = lax.axis_index("c")
        sid = lax.axis_index("s")
        wid = sid * _NC + cid
        base = wid * e_w

        if first:
            @pl.when(sid < 2)
            def _init_main():
                pltpu.sync_copy(
                    aux_hbm.at[pl.ds(sid * rows_half, rows_half)],
                    acc_sh.at[pl.ds(sid * rows_half, rows_half)],
                )

        pltpu.sync_copy(idx_hbm.at[wid], idx_v.at[pl.ds(0, n_chunks)])
        if first:
            plsc.subcore_barrier()

        bufs = (buf0, buf1)
        sems = (sem0, sem1)

        def _load(j, par):
            return pltpu.async_copy(
                msg_hbm.at[pl.ds(base + j * chunk, chunk)], bufs[par],
                sems[par],
            )

        def _process(j, par):
            @pl.when(j < n_chunks - 1)
            def _prefetch():
                _load(j + 1, 1 - par)

            pltpu.make_async_copy(
                msg_hbm.at[pl.ds(base + j * chunk, chunk)], bufs[par],
                sems[par],
            ).wait()
            pltpu.sync_copy(bufs[par], acc_sh.at[idx_v.at[j]], add=True)

        _load(0, 0)

        def body(j, carry):
            @pl.when(j % 2 == 0)
            def _even():
                _process(j, 0)

            @pl.when(j % 2 == 1)
            def _odd():
                _process(j, 1)

            return carry

        lax.fori_loop(0, n_chunks, body, 0)

        if last:
            plsc.subcore_barrier()

            @pl.when(sid < 2)
            def _publish():
                pltpu.sync_copy(
                    acc_sh.at[pl.ds(sid * rows_half, rows_half)],
                    out_hbm.at[
                        pl.ds(cid * n_nodes + sid * rows_half, rows_half)],
                )

    return k(msg, neigh3, aux)


def _combine_body(p_ref, out_ref):
    out_ref[...] = p_ref[0] + p_ref[1]


def _combine(partials, n_nodes, S):
    BR = 2000
    grid = (n_nodes // BR,)
    return pl.pallas_call(
        _combine_body,
        grid=grid,
        in_specs=[pl.BlockSpec((2, BR, S), lambda i: (0, i, 0))],
        out_specs=pl.BlockSpec((BR, S), lambda i: (i, 0)),
        out_shape=jax.ShapeDtypeStruct((n_nodes, S), jnp.float32),
    )(partials)


def kernel(pair, pair_update, neighbours, pair_mask, distance, W):
    N, K, D = pair.shape
    S = W.shape[1]
    M = N * K
    BM = 12800
    chunk = 80
    node_splits = (3200, 3200, 3600)
    H = len(node_splits)

    pairf = pair.reshape(M, D)
    updf = pair_update.reshape(M, S)
    dist2 = distance[:, None]

    idx = jnp.where(pair_mask, neighbours.astype(jnp.int32), N)

    max_chunks = max(node_splits) // chunk
    aux = jnp.zeros((N, S), jnp.float32)
    n0 = 0
    for h, Nh in enumerate(node_splits):
        Mh = Nh * K
        e_w = Mh // _NW
        msg = _compute_message(
            pairf, updf, dist2, W, K, BM, Mh, n0 * K // BM)
        neigh3 = lax.dynamic_slice_in_dim(idx, n0, Nh, 0).reshape(
            _NW, e_w // chunk, chunk)
        aux = _scatter_pass(
            msg, neigh3, aux, N, chunk, max_chunks,
            first=(h == 0), last=(h == H - 1))
        n0 += Nh

    return _combine(aux.reshape(2, N, S), N, S)


# speedup vs baseline: 1.0087x; 1.0087x over previous
import functools

import jax
import jax.numpy as jnp
from jax import lax
from jax.experimental import pallas as pl
from jax.experimental.pallas import tpu as pltpu
from jax.experimental.pallas import tpu_sc as plsc

_R_MAX = 6.0
_P = 3.0

_NC = 2
_NS = 16
_NW = _NC * _NS


def _message_body(w_ref, pair_ref, upd_ref, dist_ref, out_ref):
    bn = dist_ref.shape[0]
    k = pair_ref.shape[0] // bn
    z = jnp.dot(pair_ref[...], w_ref[...], preferred_element_type=jnp.float32)
    gate = 1.0 / (1.0 + jnp.exp(-z))
    d = dist_ref[...]
    t = d * (1.0 / _R_MAX)
    t_p = t * t * t
    env = (
        1.0
        - (_P + 1.0) * (_P + 2.0) / 2.0 * t_p
        + _P * (_P + 2.0) * t_p * t
        - _P * (_P + 1.0) / 2.0 * t_p * t * t
    )
    env = jnp.where(d < _R_MAX, env, 0.0)
    env_rows = lax.broadcast_in_dim(env, (bn, k, 1), (0, 2)).reshape(bn * k, 1)
    out_ref[...] = gate * upd_ref[...] * env_rows


def _compute_message(pairf, updf, dist2, W, K, BM, m_out, off_blocks):
    D = pairf.shape[1]
    S = W.shape[1]
    BN = BM // K
    grid = (m_out // BM,)
    return pl.pallas_call(
        _message_body,
        grid=grid,
        in_specs=[
            pl.BlockSpec((D, S), lambda i: (0, 0)),
            pl.BlockSpec((BM, D), lambda i: (i + off_blocks, 0)),
            pl.BlockSpec((BM, S), lambda i: (i + off_blocks, 0)),
            pl.BlockSpec((BN, 1), lambda i: (i + off_blocks, 0)),
        ],
        out_specs=pl.BlockSpec((BM, S), lambda i: (i, 0)),
        out_shape=jax.ShapeDtypeStruct((m_out, S), jnp.float32),
        compiler_params=pltpu.CompilerParams(
            dimension_semantics=("arbitrary",),
        ),
    )(W, pairf, updf, dist2)


def _scatter_pass(msg, neigh3, aux, n_nodes, chunk, max_chunks, first, last):
    M, S = msg.shape
    n_acc = n_nodes + 8
    e_w = M // _NW
    n_chunks = e_w // chunk
    rows_half = n_nodes // 2

    mesh = plsc.VectorSubcoreMesh(
        core_axis_name="c", subcore_axis_name="s",
        num_cores=_NC, num_subcores=_NS,
    )

    if last:
        out_type = jax.ShapeDtypeStruct((_NC * n_nodes, S), jnp.float32)
    else:
        out_type = jax.ShapeDtypeStruct((8, S), jnp.float32)

    @functools.partial(
        pl.kernel,
        out_type=out_type,
        mesh=mesh,
        scratch_types=[
            pltpu.VMEM_SHARED((n_acc, S), jnp.float32),
            pltpu.VMEM((max_chunks, chunk), jnp.int32),
            pltpu.VMEM((chunk, S), jnp.float32),
            pltpu.VMEM((chunk, S), jnp.float32),
            pltpu.SemaphoreType.DMA,
            pltpu.SemaphoreType.DMA,
        ],
    )
    def k(msg_hbm, idx_hbm, aux_hbm, out_hbm, acc_sh, idx_v, buf0, buf1,
          sem0, sem1):
        cid = lax.axis_index("c")
        sid = lax.axis_index("s")
        wid = sid * _NC + cid
        base = wid * e_w

        if first:
            @pl.when(sid < 2)
            def _init_main():
                pltpu.sync_copy(
                    aux_hbm.at[pl.ds(sid * rows_half, rows_half)],
                    acc_sh.at[pl.ds(sid * rows_half, rows_half)],
                )

        pltpu.sync_copy(idx_hbm.at[wid], idx_v.at[pl.ds(0, n_chunks)])
        if first:
            plsc.subcore_barrier()

        bufs = (buf0, buf1)
        sems = (sem0, sem1)

        def _load(j, par):
            return pltpu.async_copy(
                msg_hbm.at[pl.ds(base + j * chunk, chunk)], bufs[par],
                sems[par],
            )

        def _process(j, par):
            @pl.when(j < n_chunks - 1)
            def _prefetch():
                _load(j + 1, 1 - par)

            pltpu.make_async_copy(
                msg_hbm.at[pl.ds(base + j * chunk, chunk)], bufs[par],
                sems[par],
            ).wait()
            pltpu.sync_copy(bufs[par], acc_sh.at[idx_v.at[j]], add=True)

        _load(0, 0)

        def body(j, carry):
            @pl.when(j % 2 == 0)
            def _even():
                _process(j, 0)

            @pl.when(j % 2 == 1)
            def _odd():
                _process(j, 1)

            return carry

        lax.fori_loop(0, n_chunks, body, 0)

        if last:
            plsc.subcore_barrier()

            @pl.when(sid < 2)
            def _publish():
                pltpu.sync_copy(
                    acc_sh.at[pl.ds(sid * rows_half, rows_half)],
                    out_hbm.at[
                        pl.ds(cid * n_nodes + sid * rows_half, rows_half)],
                )

    return k(msg, neigh3, aux)


def _combine_body(p_ref, out_ref):
    out_ref[...] = p_ref[0] + p_ref[1]


def _combine(partials, n_nodes, S):
    BR = 2000
    grid = (n_nodes // BR,)
    return pl.pallas_call(
        _combine_body,
        grid=grid,
        in_specs=[pl.BlockSpec((2, BR, S), lambda i: (0, i, 0))],
        out_specs=pl.BlockSpec((BR, S), lambda i: (i, 0)),
        out_shape=jax.ShapeDtypeStruct((n_nodes, S), jnp.float32),
    )(partials)


def kernel(pair, pair_update, neighbours, pair_mask, distance, W):
    N, K, D = pair.shape
    S = W.shape[1]
    M = N * K
    BM = 12800
    chunk = 80
    node_splits = (800, 3200, 3200, 2800)
    H = len(node_splits)

    pairf = pair.reshape(M, D)
    updf = pair_update.reshape(M, S)
    dist2 = distance[:, None]

    idx = jnp.where(pair_mask, neighbours.astype(jnp.int32), N)

    max_chunks = max(node_splits) // chunk
    aux = jnp.zeros((N, S), jnp.float32)
    n0 = 0
    for h, Nh in enumerate(node_splits):
        Mh = Nh * K
        e_w = Mh // _NW
        msg = _compute_message(
            pairf, updf, dist2, W, K, BM, Mh, n0 * K // BM)
        neigh3 = lax.dynamic_slice_in_dim(idx, n0, Nh, 0).reshape(
            _NW, e_w // chunk, chunk)
        aux = _scatter_pass(
            msg, neigh3, aux, N, chunk, max_chunks,
            first=(h == 0), last=(h == H - 1))
        n0 += Nh

    return _combine(aux.reshape(2, N, S), N, S)
